# baseline (speedup 1.0000x reference)
import jax
import jax.numpy as jnp
import numpy as np
from jax.experimental import pallas as pl
from jax.experimental.pallas import tpu as pltpu

B = 16
T = 128
V = 25
M = 2
NUM_CLASSES = 27

SK_TG = 5
SK_SLOTS = 260
SK_GROUPS = SK_SLOTS // SK_TG
SK_COLS = SK_GROUPS * 128
SK_VALID = M * T * V

IMU_TG = 16
IMU_GROUPS = T // IMU_TG
IMU_COLS = IMU_GROUPS * 128
IMU_VALID = T * 6

BF = jnp.bfloat16

SPP = 4
PROGS = B // SPP


def _spatial(x, a, groups):
    return jnp.concatenate(
        [jax.lax.dot_general(x[:, g * 128:(g + 1) * 128], a,
                             (((1,), (0,)), ((), ())),
                             preferred_element_type=jnp.float32)
         .astype(BF)
         for g in range(groups)], axis=1)


def _channel_relu(wt, x):
    z = jax.lax.dot_general(wt, x, (((1,), (0,)), ((), ())),
                            preferred_element_type=jnp.float32)
    return jnp.maximum(z.astype(BF), BF(0.0))


def _fused_kernel(xs_ref, ys_ref, *refs):
    a_sk = refs[0:10]
    w_sk = refs[10:20]
    a_imu = refs[20]
    w_imu = refs[21:26]
    wtop_ref, wbot_ref, b_ref, out_ref = refs[26:30]

    x = xs_ref[0]
    for l in range(10):
        x = _channel_relu(w_sk[l][...], _spatial(x, a_sk[l][...],
                                                 SPP * SK_GROUPS))

    y = ys_ref[0]
    a = a_imu[...]
    for l in range(5):
        y = _channel_relu(w_imu[l][...], _spatial(y, a, SPP * IMU_GROUPS))

    top = jax.lax.dot_general(wtop_ref[...], x, (((1,), (0,)), ((), ())),
                              preferred_element_type=jnp.float32)
    bot = jax.lax.dot_general(wbot_ref[...], y, (((1,), (0,)), ((), ())),
                              preferred_element_type=jnp.float32)
    for s in range(SPP):
        out = (jnp.sum(top[:, s * SK_COLS:(s + 1) * SK_COLS], axis=1)
               * (1.0 / SK_VALID)
               + jnp.sum(bot[:, s * IMU_COLS:(s + 1) * IMU_COLS], axis=1)
               * (1.0 / IMU_VALID) + b_ref[0, :])
        out_ref[0, s, :] = out


def _full(shape):
    ndim = len(shape)
    return pl.BlockSpec(shape, lambda *_: (0,) * ndim)


def kernel(skeleton, inertial, A_sk, A_imu, Ws_sk, Bs_sk, Ws_imu, W_fc, b_fc):
    f32 = jnp.float32

    eye5 = jnp.eye(SK_TG, dtype=f32)
    a_sk_packed = []
    for Badp in Bs_sk:
        ahat = A_sk + Badp
        a5 = jnp.kron(eye5, ahat)
        a_sk_packed.append(jnp.pad(a5, ((0, 3), (0, 3))).astype(BF))
    wt_sk = [w.T.astype(BF) for w in Ws_sk]

    eye16 = jnp.eye(IMU_TG, dtype=f32)
    a_imu_packed = jnp.pad(jnp.kron(eye16, A_imu),
                           ((0, 32), (0, 32))).astype(BF)
    wt_imu = [w.T.astype(BF) for w in Ws_imu]

    wtop = W_fc[:256].T.astype(BF)
    wbot = W_fc[256:].T.astype(BF)
    b2 = b_fc.reshape(1, NUM_CLASSES)

    xs = jnp.transpose(skeleton, (0, 1, 4, 2, 3))
    xs = xs.reshape(B, 3, M * T, V)
    xs = jnp.pad(xs, ((0, 0), (0, 0), (0, SK_SLOTS - M * T), (0, 0)))
    xs = xs.reshape(B, 3, SK_GROUPS, SK_TG * V)
    xs = jnp.pad(xs, ((0, 0), (0, 0), (0, 0), (0, 3)))
    xs = xs.reshape(PROGS, SPP, 3, SK_COLS).transpose(0, 2, 1, 3)
    xs = xs.reshape(PROGS, 3, SPP * SK_COLS).astype(BF)

    ys = jnp.transpose(inertial, (0, 2, 1))
    ys = ys.reshape(B, IMU_GROUPS, IMU_TG * 6)
    ys = jnp.pad(ys, ((0, 0), (0, 0), (0, 32)))
    ys = ys.reshape(PROGS, 1, SPP * IMU_COLS).astype(BF)

    in_specs = ([pl.BlockSpec((1, 3, SPP * SK_COLS), lambda i: (i, 0, 0)),
                 pl.BlockSpec((1, 1, SPP * IMU_COLS), lambda i: (i, 0, 0))]
                + [_full((128, 128)) for _ in range(10)]
                + [_full(w.shape) for w in wt_sk]
                + [_full((128, 128))]
                + [_full(w.shape) for w in wt_imu]
                + [_full((NUM_CLASSES, 256)), _full((NUM_CLASSES, 256)),
                   _full((1, NUM_CLASSES))])
    out = pl.pallas_call(
        _fused_kernel,
        grid=(PROGS,),
        in_specs=in_specs,
        out_specs=pl.BlockSpec((1, SPP, NUM_CLASSES), lambda i: (i, 0, 0)),
        out_shape=jax.ShapeDtypeStruct((PROGS, SPP, NUM_CLASSES), f32),
        compiler_params=pltpu.CompilerParams(
            dimension_semantics=("parallel",)),
    )(xs, ys, *a_sk_packed, *wt_sk, a_imu_packed, *wt_imu, wtop, wbot, b2)
    return out.reshape(B, NUM_CLASSES)


# speedup vs baseline: 1.2752x; 1.2752x over previous
import jax
import jax.numpy as jnp
import numpy as np
from jax.experimental import pallas as pl
from jax.experimental.pallas import tpu as pltpu

B = 16
T = 128
V = 25
M = 2
NUM_CLASSES = 27

SK_TG = 5
SK_SLOTS = 260
SK_GROUPS = SK_SLOTS // SK_TG
SK_COLS = SK_GROUPS * 128
SK_VALID = M * T * V

IMU_TG = 16
IMU_GROUPS = T // IMU_TG
IMU_COLS = IMU_GROUPS * 128
IMU_VALID = T * 6

BF = jnp.bfloat16

SPP = 2
PROGS = B // SPP


def _spatial(x, a, groups):
    return jnp.concatenate(
        [jax.lax.dot_general(x[:, g * 128:(g + 1) * 128], a,
                             (((1,), (0,)), ((), ())),
                             preferred_element_type=jnp.float32)
         .astype(BF)
         for g in range(groups)], axis=1)


def _channel_relu(wt, x):
    z = jax.lax.dot_general(wt, x, (((1,), (0,)), ((), ())),
                            preferred_element_type=jnp.float32)
    return jnp.maximum(z.astype(BF), BF(0.0))


def _fused_kernel(xs_ref, ys_ref, *refs):
    a_sk = refs[0:10]
    w_sk = refs[10:20]
    a_imu = refs[20]
    w_imu = refs[21:26]
    wtop_ref, wbot_ref, b_ref, out_ref = refs[26:30]

    x = xs_ref[0]
    for l in range(10):
        x = _channel_relu(w_sk[l][...], _spatial(x, a_sk[l][...],
                                                 SPP * SK_GROUPS))

    y = ys_ref[0]
    a = a_imu[...]
    for l in range(5):
        y = _channel_relu(w_imu[l][...], _spatial(y, a, SPP * IMU_GROUPS))

    top = jax.lax.dot_general(wtop_ref[...], x, (((1,), (0,)), ((), ())),
                              preferred_element_type=jnp.float32)
    bot = jax.lax.dot_general(wbot_ref[...], y, (((1,), (0,)), ((), ())),
                              preferred_element_type=jnp.float32)
    for s in range(SPP):
        out = (jnp.sum(top[:, s * SK_COLS:(s + 1) * SK_COLS], axis=1)
               * (1.0 / SK_VALID)
               + jnp.sum(bot[:, s * IMU_COLS:(s + 1) * IMU_COLS], axis=1)
               * (1.0 / IMU_VALID) + b_ref[0, :])
        out_ref[0, s, :] = out


def _full(shape):
    ndim = len(shape)
    return pl.BlockSpec(shape, lambda *_: (0,) * ndim)


def kernel(skeleton, inertial, A_sk, A_imu, Ws_sk, Bs_sk, Ws_imu, W_fc, b_fc):
    f32 = jnp.float32

    eye5 = jnp.eye(SK_TG, dtype=f32)
    a_sk_packed = []
    for Badp in Bs_sk:
        ahat = A_sk + Badp
        a5 = jnp.kron(eye5, ahat)
        a_sk_packed.append(jnp.pad(a5, ((0, 3), (0, 3))).astype(BF))
    wt_sk = [w.T.astype(BF) for w in Ws_sk]

    eye16 = jnp.eye(IMU_TG, dtype=f32)
    a_imu_packed = jnp.pad(jnp.kron(eye16, A_imu),
                           ((0, 32), (0, 32))).astype(BF)
    wt_imu = [w.T.astype(BF) for w in Ws_imu]

    wtop = W_fc[:256].T.astype(BF)
    wbot = W_fc[256:].T.astype(BF)
    b2 = b_fc.reshape(1, NUM_CLASSES)

    xs = jnp.transpose(skeleton, (0, 1, 4, 2, 3))
    xs = xs.reshape(B, 3, M * T, V)
    xs = jnp.pad(xs, ((0, 0), (0, 0), (0, SK_SLOTS - M * T), (0, 0)))
    xs = xs.reshape(B, 3, SK_GROUPS, SK_TG * V)
    xs = jnp.pad(xs, ((0, 0), (0, 0), (0, 0), (0, 3)))
    xs = xs.reshape(PROGS, SPP, 3, SK_COLS).transpose(0, 2, 1, 3)
    xs = xs.reshape(PROGS, 3, SPP * SK_COLS).astype(BF)

    ys = jnp.transpose(inertial, (0, 2, 1))
    ys = ys.reshape(B, IMU_GROUPS, IMU_TG * 6)
    ys = jnp.pad(ys, ((0, 0), (0, 0), (0, 32)))
    ys = ys.reshape(PROGS, 1, SPP * IMU_COLS).astype(BF)

    in_specs = ([pl.BlockSpec((1, 3, SPP * SK_COLS), lambda i: (i, 0, 0)),
                 pl.BlockSpec((1, 1, SPP * IMU_COLS), lambda i: (i, 0, 0))]
                + [_full((128, 128)) for _ in range(10)]
                + [_full(w.shape) for w in wt_sk]
                + [_full((128, 128))]
                + [_full(w.shape) for w in wt_imu]
                + [_full((NUM_CLASSES, 256)), _full((NUM_CLASSES, 256)),
                   _full((1, NUM_CLASSES))])
    out = pl.pallas_call(
        _fused_kernel,
        grid=(PROGS,),
        in_specs=in_specs,
        out_specs=pl.BlockSpec((1, SPP, NUM_CLASSES), lambda i: (i, 0, 0)),
        out_shape=jax.ShapeDtypeStruct((PROGS, SPP, NUM_CLASSES), f32),
        compiler_params=pltpu.CompilerParams(
            dimension_semantics=("arbitrary",),
            vmem_limit_bytes=100 * 1024 * 1024),
    )(xs, ys, *a_sk_packed, *wt_sk, a_imu_packed, *wt_imu, wtop, wbot, b2)
    return out.reshape(B, NUM_CLASSES)
